# baseline (speedup 1.0000x reference)
import functools
import math

import jax
import jax.numpy as jnp
from jax import lax
from jax.experimental import pallas as pl
from jax.experimental.pallas import tpu as pltpu
from jax.experimental.pallas import tpu_sc as plsc

SEQ_LEN = 8192
DIM = 64
_NC, _NS = 2, 16
_NW = _NC * _NS
_B_PER_W = SEQ_LEN // _NW


def _positional_encoding():
    position = jnp.arange(SEQ_LEN, dtype=jnp.float32)[:, None]
    div_term = jnp.exp(
        jnp.arange(0, DIM, 2, dtype=jnp.float32) * (-math.log(10000.0) / DIM)
    )
    pe = jnp.zeros((SEQ_LEN, DIM), dtype=jnp.float32)
    pe = pe.at[:, 0::2].set(jnp.sin(position * div_term))
    pe = pe.at[:, 1::2].set(jnp.cos(position * div_term))
    return pe


_mesh = plsc.VectorSubcoreMesh(core_axis_name="c", subcore_axis_name="s")


@functools.partial(
    pl.kernel,
    mesh=_mesh,
    out_type=jax.ShapeDtypeStruct((SEQ_LEN, DIM), jnp.float32),
    scratch_types=[
        pltpu.VMEM((_B_PER_W,), jnp.int32),
        pltpu.VMEM((_B_PER_W, DIM), jnp.float32),
        pltpu.SemaphoreType.DMA,
    ],
    compiler_params=pltpu.CompilerParams(use_tc_tiling_on_sc=False),
)
def _sc_gather(table_hbm, idx_hbm, out_hbm, idx_v, rows_v, sem):
    wid = lax.axis_index("s") * _NC + lax.axis_index("c")
    base = wid * _B_PER_W
    pltpu.sync_copy(idx_hbm.at[pl.ds(base, _B_PER_W)], idx_v)
    pltpu.async_copy(table_hbm.at[idx_v], rows_v, sem).wait()
    pltpu.sync_copy(rows_v, out_hbm.at[pl.ds(base, _B_PER_W)])


def _tc_add(x_ref, pe_ref, o_ref):
    o_ref[...] = x_ref[...] + pe_ref[...]


def kernel(indices, table):
    idx = indices.astype(jnp.int32)
    gathered = _sc_gather(table, idx)
    pe = _positional_encoding()
    out = pl.pallas_call(
        _tc_add,
        out_shape=jax.ShapeDtypeStruct((SEQ_LEN, DIM), jnp.float32),
    )(gathered, pe)
    return out[None, :, :]
